# baseline (speedup 1.0000x reference)
import functools

import jax
import jax.numpy as jnp
from jax import lax
from jax.experimental import pallas as pl
from jax.experimental.pallas import tpu as pltpu
from jax.experimental.pallas import tpu_sc as plsc

N = 10000
E = 320000
F = 128
FH = F // 2
NP = 10240
NC = 2
NS = 16
C = 128
BCH = 8
NB = 20
CH = NB * BCH
EPT = CH * C
E_PAD = NS * EPT
NBUF = 4
RPT = NP // NS
ZR = 64
DCH = CH // NC

_mesh = plsc.VectorSubcoreMesh(core_axis_name="c", subcore_axis_name="s")


@functools.partial(
    pl.kernel,
    out_type=jax.ShapeDtypeStruct((NC, NP), jnp.float32),
    mesh=_mesh,
    scratch_types=[
        pltpu.VMEM((DCH, C), jnp.int32),
        pltpu.VMEM((C,), jnp.float32),
        pltpu.VMEM((RPT,), jnp.float32),
        pltpu.VMEM_SHARED((NP,), jnp.float32),
    ],
)
def _deg_kernel(colp_hbm, out_hbm, idx_v, ones_v, zer_v, deg_sh):
    cid = lax.axis_index("c")
    sid = lax.axis_index("s")

    for k in range(C // 16):
        ones_v[pl.ds(16 * k, 16)] = jnp.ones((16,), jnp.float32)

    def _zfill(i, carry):
        zer_v[pl.ds(i * 16, 16)] = jnp.zeros((16,), jnp.float32)
        return carry

    lax.fori_loop(0, RPT // 16, _zfill, 0)
    pltpu.sync_copy(zer_v, deg_sh.at[pl.ds(sid * RPT, RPT)])
    pltpu.sync_copy(colp_hbm.at[sid, cid], idx_v)
    plsc.subcore_barrier()

    def _acc(j, carry):
        pltpu.sync_copy(ones_v, deg_sh.at[idx_v.at[j]], add=True)
        return carry

    lax.fori_loop(0, DCH, _acc, 0)
    plsc.subcore_barrier()
    pltpu.sync_copy(
        deg_sh.at[pl.ds(sid * RPT, RPT)],
        out_hbm.at[cid, pl.ds(sid * RPT, RPT)],
    )


@functools.partial(
    pl.kernel,
    out_type=jax.ShapeDtypeStruct((NC, NP, FH), jnp.float32),
    mesh=_mesh,
    compiler_params=pltpu.CompilerParams(use_tc_tiling_on_sc=False),
    scratch_types=[
        pltpu.VMEM((2, BCH, C), jnp.int32),
        pltpu.VMEM((CH, C), jnp.int32),
        pltpu.VMEM((NBUF, C, FH), jnp.float32),
        pltpu.VMEM_SHARED((NP, FH), jnp.float32),
        pltpu.SemaphoreType.DMA,
        pltpu.SemaphoreType.DMA,
        pltpu.SemaphoreType.DMA,
        pltpu.SemaphoreType.DMA,
        pltpu.SemaphoreType.DMA,
    ],
)
def _edge_kernel(rowp_hbm, colp_hbm, h2r_hbm, out_hbm,
                 ring_v, cidx_v, gbuf, acc_sh,
                 semg0, semg1, semg2, semg3, semi):
    cid = lax.axis_index("c")
    sid = lax.axis_index("s")
    semg = (semg0, semg1, semg2, semg3)
    h2c = h2r_hbm.at[cid]

    def _zfill(i, carry):
        for k in range(FH // 16):
            gbuf[0, i, pl.ds(16 * k, 16)] = jnp.zeros((16,), jnp.float32)
        return carry

    lax.fori_loop(0, ZR, _zfill, 0)

    def _zacc(i, carry):
        pltpu.sync_copy(gbuf.at[0, pl.ds(0, ZR)],
                        acc_sh.at[pl.ds(sid * RPT + i * ZR, ZR)])
        return carry

    lax.fori_loop(0, RPT // ZR, _zacc, 0)
    pltpu.sync_copy(colp_hbm.at[sid], cidx_v)
    pltpu.sync_copy(rowp_hbm.at[sid, 0], ring_v.at[0])
    pltpu.async_copy(rowp_hbm.at[sid, 1], ring_v.at[1], semi)
    plsc.subcore_barrier()

    for t in range(NBUF):
        pltpu.async_copy(h2c.at[ring_v.at[0, t]], gbuf.at[t], semg[t])

    def _body(bi, carry):
        for k in range(BCH):
            j = bi * BCH + k
            b = k % NBUF
            slot_j = bi % 2
            slot_j4 = (bi + (k + NBUF) // BCH) % 2
            pltpu.make_async_copy(
                h2c.at[ring_v.at[slot_j, k]], gbuf.at[b], semg[b]
            ).wait()
            pltpu.sync_copy(gbuf.at[b], acc_sh.at[cidx_v.at[j]], add=True)

            if k == BCH - NBUF - 1:
                @pl.when(bi < NB - 1)
                def _():
                    pltpu.make_async_copy(
                        rowp_hbm.at[sid, bi + 1], ring_v.at[(bi + 1) % 2], semi
                    ).wait()

            @pl.when(j + NBUF < CH)
            def _():
                pltpu.async_copy(
                    h2c.at[ring_v.at[slot_j4, (k + NBUF) % BCH]],
                    gbuf.at[b], semg[b],
                )

            if k == BCH - 1:
                @pl.when(bi < NB - 2)
                def _():
                    pltpu.async_copy(
                        rowp_hbm.at[sid, bi + 2], ring_v.at[bi % 2], semi
                    )
        return carry

    lax.fori_loop(0, NB, _body, 0)
    plsc.subcore_barrier()
    pltpu.sync_copy(
        acc_sh.at[pl.ds(sid * RPT, RPT)],
        out_hbm.at[cid, pl.ds(sid * RPT, RPT)],
    )


BM = 1000
BO = 1000


def _h2_body(x_ref, w_ref, dpt_ref, o_ref):
    dp = dpt_ref[...]
    dinv = lax.rsqrt(dp[:, 0:1] + dp[:, 1:2] + 1.0)
    h = jnp.dot(x_ref[...], w_ref[...], preferred_element_type=jnp.float32)
    h = h * dinv
    o_ref[0] = h[:, :FH]
    o_ref[1] = h[:, FH:]


_h2_matmul = pl.pallas_call(
    _h2_body,
    grid=(N // BM,),
    in_specs=[
        pl.BlockSpec((BM, F), lambda i: (i, 0)),
        pl.BlockSpec((F, F), lambda i: (0, 0)),
        pl.BlockSpec((BM, NC), lambda i: (i, 0)),
    ],
    out_specs=pl.BlockSpec((NC, BM, FH), lambda i: (0, i, 0)),
    out_shape=jax.ShapeDtypeStruct((NC, N, FH), jnp.float32),
)


def _comb_body(p_ref, h2_ref, dpt_ref, b_ref, o_ref):
    dp = dpt_ref[...]
    dinv = lax.rsqrt(dp[:, 0:1] + dp[:, 1:2] + 1.0)
    s = jnp.concatenate(
        [p_ref[0] + h2_ref[0], p_ref[1] + h2_ref[1]], axis=1)
    o_ref[...] = dinv * s + b_ref[...]


_combine = pl.pallas_call(
    _comb_body,
    grid=(N // BO,),
    in_specs=[
        pl.BlockSpec((NC, BO, FH), lambda i: (0, i, 0)),
        pl.BlockSpec((NC, BO, FH), lambda i: (0, i, 0)),
        pl.BlockSpec((BO, NC), lambda i: (i, 0)),
        pl.BlockSpec((1, F), lambda i: (0, 0)),
    ],
    out_specs=pl.BlockSpec((BO, F), lambda i: (i, 0)),
    out_shape=jax.ShapeDtypeStruct((N, F), jnp.float32),
)


@jax.jit
def kernel(x, edge_index, W, b):
    row = edge_index[0].astype(jnp.int32)
    col = edge_index[1].astype(jnp.int32)
    npad = E_PAD - E
    padg = jnp.arange(npad, dtype=jnp.int32) % (NP - N)
    pads = N + padg
    rowp = jnp.concatenate([row, padg]).reshape(NS, NB, BCH, C)
    colp = jnp.concatenate([col, pads]).reshape(NS, CH, C)
    colp_deg = colp.reshape(NS, NC, DCH, C)

    degp = _deg_kernel(colp_deg)
    degp_t = degp.T[:N]
    h2r = _h2_matmul(x, W, degp_t)
    parts = _edge_kernel(rowp, colp, h2r)
    return _combine(parts, h2r, degp_t, b.reshape(1, F))


# speedup vs baseline: 1.0682x; 1.0682x over previous
import functools

import jax
import jax.numpy as jnp
from jax import lax
from jax.experimental import pallas as pl
from jax.experimental.pallas import tpu as pltpu
from jax.experimental.pallas import tpu_sc as plsc

N = 10000
E = 320000
F = 128
NP = 10240
NC = 2
NS = 16
NW = NC * NS
C = 128
BCH = 8
NB = 10
CH = NB * BCH
E_PAD = NW * CH * C
RPT = NP // NS
ZR = 64

_mesh = plsc.VectorSubcoreMesh(core_axis_name="c", subcore_axis_name="s")


@functools.partial(
    pl.kernel,
    out_type=jax.ShapeDtypeStruct((NC, NP), jnp.float32),
    mesh=_mesh,
    scratch_types=[
        pltpu.VMEM((CH, C), jnp.int32),
        pltpu.VMEM((C,), jnp.float32),
        pltpu.VMEM((RPT,), jnp.float32),
        pltpu.VMEM_SHARED((NP,), jnp.float32),
        pltpu.SemaphoreType.DMA,
    ],
)
def _deg_kernel(colp_hbm, out_hbm, idx_v, ones_v, zer_v, deg_sh, semd):
    cid = lax.axis_index("c")
    sid = lax.axis_index("s")
    wid = sid * NC + cid

    for k in range(C // 16):
        ones_v[pl.ds(16 * k, 16)] = jnp.ones((16,), jnp.float32)

    def _zfill(i, carry):
        zer_v[pl.ds(i * 16, 16)] = jnp.zeros((16,), jnp.float32)
        return carry

    lax.fori_loop(0, RPT // 16, _zfill, 0)
    pltpu.sync_copy(zer_v, deg_sh.at[pl.ds(sid * RPT, RPT)])
    pltpu.sync_copy(colp_hbm.at[wid], idx_v)
    plsc.subcore_barrier()

    def _acc(r, carry):
        for t in range(8):
            pltpu.async_copy(ones_v, deg_sh.at[idx_v.at[r * 8 + t]], semd,
                             add=True)
        for t in range(8):
            pltpu.make_async_copy(
                ones_v, deg_sh.at[idx_v.at[r * 8 + t]], semd).wait()
        return carry

    lax.fori_loop(0, CH // 8, _acc, 0)
    plsc.subcore_barrier()
    pltpu.sync_copy(
        deg_sh.at[pl.ds(sid * RPT, RPT)],
        out_hbm.at[cid, pl.ds(sid * RPT, RPT)],
    )


@functools.partial(
    pl.kernel,
    out_type=jax.ShapeDtypeStruct((NC, NP, F), jnp.float32),
    mesh=_mesh,
    scratch_types=[
        pltpu.VMEM((2, BCH, C), jnp.int32),
        pltpu.VMEM((CH, C), jnp.int32),
        pltpu.VMEM((2, C, F), jnp.float32),
        pltpu.VMEM((16, F), jnp.float32),
        pltpu.VMEM_SHARED((NP, F), jnp.float32),
        pltpu.SemaphoreType.DMA,
        pltpu.SemaphoreType.DMA,
        pltpu.SemaphoreType.DMA,
        pltpu.SemaphoreType.DMA,
    ],
)
def _edge_kernel(rowp_hbm, colp_hbm, h2_hbm, out_hbm,
                 ring_v, cidx_v, gbuf, zbuf, acc_sh, semg0, semg1, semi, semz):
    cid = lax.axis_index("c")
    sid = lax.axis_index("s")
    wid = sid * NC + cid
    semg = (semg0, semg1)

    pltpu.sync_copy(colp_hbm.at[wid], cidx_v)
    pltpu.sync_copy(rowp_hbm.at[wid, 0], ring_v.at[0])
    pltpu.async_copy(rowp_hbm.at[wid, 1], ring_v.at[1], semi)
    pltpu.async_copy(h2_hbm.at[ring_v.at[0, 0]], gbuf.at[0], semg0)
    pltpu.async_copy(h2_hbm.at[ring_v.at[0, 1]], gbuf.at[1], semg1)

    for i in range(16):
        for k in range(F // 16):
            zbuf[i, pl.ds(16 * k, 16)] = jnp.zeros((16,), jnp.float32)

    def _zacc(r, carry):
        for t in range(8):
            pltpu.async_copy(
                zbuf, acc_sh.at[pl.ds(sid * RPT + (r * 8 + t) * 16, 16)], semz)
        for t in range(8):
            pltpu.make_async_copy(
                zbuf, acc_sh.at[pl.ds(sid * RPT + (r * 8 + t) * 16, 16)],
                semz).wait()
        return carry

    lax.fori_loop(0, RPT // 128, _zacc, 0)
    plsc.subcore_barrier()

    def _body(bi, carry):
        for k in range(BCH):
            j = bi * BCH + k
            b = k % 2
            slot_j = bi % 2
            slot_j2 = (bi + (k + 2) // BCH) % 2
            pltpu.make_async_copy(
                h2_hbm.at[ring_v.at[slot_j, k]], gbuf.at[b], semg[b]
            ).wait()
            pltpu.sync_copy(gbuf.at[b], acc_sh.at[cidx_v.at[j]], add=True)

            if k == BCH - 2:
                @pl.when(bi < NB - 1)
                def _():
                    pltpu.make_async_copy(
                        rowp_hbm.at[wid, bi + 1], ring_v.at[(bi + 1) % 2], semi
                    ).wait()

            @pl.when(j + 2 < CH)
            def _():
                pltpu.async_copy(
                    h2_hbm.at[ring_v.at[slot_j2, (k + 2) % BCH]],
                    gbuf.at[b], semg[b],
                )

            if k == BCH - 1:
                @pl.when(bi < NB - 2)
                def _():
                    pltpu.async_copy(
                        rowp_hbm.at[wid, bi + 2], ring_v.at[bi % 2], semi
                    )
        return carry

    lax.fori_loop(0, NB, _body, 0)
    plsc.subcore_barrier()
    pltpu.sync_copy(
        acc_sh.at[pl.ds(sid * RPT, RPT)],
        out_hbm.at[cid, pl.ds(sid * RPT, RPT)],
    )


BM = 1000
BO = 1000


def _h2_body(x_ref, w_ref, dpt_ref, o_ref):
    dp = dpt_ref[...]
    dinv = lax.rsqrt(dp[:, 0:1] + dp[:, 1:2] + 1.0)
    h = jnp.dot(x_ref[...], w_ref[...], preferred_element_type=jnp.float32)
    o_ref[...] = h * dinv


_h2_matmul = pl.pallas_call(
    _h2_body,
    grid=(N // BM,),
    in_specs=[
        pl.BlockSpec((BM, F), lambda i: (i, 0)),
        pl.BlockSpec((F, F), lambda i: (0, 0)),
        pl.BlockSpec((BM, NC), lambda i: (i, 0)),
    ],
    out_specs=pl.BlockSpec((BM, F), lambda i: (i, 0)),
    out_shape=jax.ShapeDtypeStruct((N, F), jnp.float32),
)


def _comb_body(p_ref, h2_ref, dpt_ref, b_ref, o_ref):
    dp = dpt_ref[...]
    dinv = lax.rsqrt(dp[:, 0:1] + dp[:, 1:2] + 1.0)
    o_ref[...] = dinv * (p_ref[0] + p_ref[1] + h2_ref[...]) + b_ref[...]


_combine = pl.pallas_call(
    _comb_body,
    grid=(N // BO,),
    in_specs=[
        pl.BlockSpec((NC, BO, F), lambda i: (0, i, 0)),
        pl.BlockSpec((BO, F), lambda i: (i, 0)),
        pl.BlockSpec((BO, NC), lambda i: (i, 0)),
        pl.BlockSpec((1, F), lambda i: (0, 0)),
    ],
    out_specs=pl.BlockSpec((BO, F), lambda i: (i, 0)),
    out_shape=jax.ShapeDtypeStruct((N, F), jnp.float32),
)


@jax.jit
def kernel(x, edge_index, W, b):
    row = edge_index[0].astype(jnp.int32)
    col = edge_index[1].astype(jnp.int32)
    npad = E_PAD - E
    padg = jnp.arange(npad, dtype=jnp.int32) % (NP - N)
    pads = N + padg
    rowp = jnp.concatenate([row, padg]).reshape(NW, NB, BCH, C)
    colp = jnp.concatenate([col, pads]).reshape(NW, CH, C)

    degp = _deg_kernel(colp)
    degp_t = degp.T[:N]
    h2 = _h2_matmul(x, W, degp_t)
    parts = _edge_kernel(rowp, colp, h2)
    return _combine(parts, h2, degp_t, b.reshape(1, F))
